# Initial kernel scaffold; baseline (speedup 1.0000x reference)
import jax
import jax.numpy as jnp
from jax.experimental import pallas as pl


def kernel(x, edge_index, edge_attr, global_state, edge_out, origin_x, atom_index, e_idx, Wg1, bg1, Wg2, bg2, Wg3, bg3, Wneg, bneg, Wroot, broot, Wedge, bedge, We, be):
    raise NotImplementedError("write your pallas kernel here")


# speedup vs baseline: 1.3692x; 1.3692x over previous
import functools

import jax
import jax.numpy as jnp
from jax.experimental import pallas as pl
from jax.experimental.pallas import tpu as pltpu

N = 10000
E = 320000
IN_CH = 128
OUT_CH = 128
EDGE_DIM = 48
STATE = 5

_NODE_BLK = 1024
_EDGE_BLK = 4096


def _elu(v):
    return jnp.where(v > 0, v, jnp.exp(jnp.minimum(v, 0.0)) - 1.0)


def _node_dense_kernel(x_ref, gstate_ref, ox_ref,
                       Wg1_ref, bg1_ref, Wg2_ref, bg2_ref, Wg3_ref, bg3_ref,
                       Wneg_x_ref, Wneg_g_ref, Wroot_x_ref, Wroot_g_ref, broot_ref,
                       Wedge_ox_ref, Wedge_g_ref, bedge_ref,
                       gs_out_ref, node_proj_ref, root_ref, oc_proj_ref):
    gstate = gstate_ref[...]
    h = jnp.maximum(gstate @ Wg1_ref[...] + bg1_ref[...], 0.0)
    h = jnp.maximum(h @ Wg2_ref[...] + bg2_ref[...], 0.0)
    gs = h @ Wg3_ref[...] + bg3_ref[...]
    gs_out_ref[...] = gs
    x = x_ref[...]
    node_proj_ref[...] = x @ Wneg_x_ref[...] + gs @ Wneg_g_ref[...]
    root_ref[...] = _elu(x @ Wroot_x_ref[...] + gs @ Wroot_g_ref[...] + broot_ref[...])
    oc_proj_ref[...] = (ox_ref[...] @ Wedge_ox_ref[...] + gs @ Wedge_g_ref[...]
                        + bedge_ref[...])


def _edge_dense_kernel(ea_ref, eo_ref,
                       Wneg_e_ref, bneg_ref, Wedge_eo_ref, We_ref, be_ref,
                       ea_proj_ref, eo_proj_ref, ee_ref):
    eo = eo_ref[...]
    ea_proj_ref[...] = ea_ref[...] @ Wneg_e_ref[...] + bneg_ref[...]
    eo_proj_ref[...] = eo @ Wedge_eo_ref[...]
    ee_ref[...] = _elu(eo @ We_ref[...] + be_ref[...])


def _row_spec(blk, ncols):
    return pl.BlockSpec((blk, ncols), lambda i: (i, 0))


def _full_spec(arr):
    return pl.BlockSpec(arr.shape, lambda i: (0,) * arr.ndim)


def _node_dense(x, global_state, origin_x, Wg1, bg1, Wg2, bg2, Wg3, bg3,
                Wneg, bneg, Wroot, broot, Wedge, bedge):
    grid = (pl.cdiv(N, _NODE_BLK),)
    out_shape = (
        jax.ShapeDtypeStruct((N, STATE), jnp.float32),
        jax.ShapeDtypeStruct((N, IN_CH), jnp.float32),
        jax.ShapeDtypeStruct((N, OUT_CH), jnp.float32),
        jax.ShapeDtypeStruct((N, EDGE_DIM), jnp.float32),
    )
    weights = (
        Wg1, bg1.reshape(1, -1), Wg2, bg2.reshape(1, -1), Wg3, bg3.reshape(1, -1),
        Wneg[:IN_CH], Wneg[IN_CH:IN_CH + STATE],
        Wroot[:IN_CH], Wroot[IN_CH:], broot.reshape(1, -1),
        Wedge[EDGE_DIM:EDGE_DIM + IN_CH], Wedge[EDGE_DIM + IN_CH:],
        bedge.reshape(1, -1))
    in_specs = [
        _row_spec(_NODE_BLK, IN_CH),
        _row_spec(_NODE_BLK, STATE),
        _row_spec(_NODE_BLK, IN_CH),
    ] + [_full_spec(w) for w in weights]
    out_specs = (
        _row_spec(_NODE_BLK, STATE),
        _row_spec(_NODE_BLK, IN_CH),
        _row_spec(_NODE_BLK, OUT_CH),
        _row_spec(_NODE_BLK, EDGE_DIM),
    )
    return pl.pallas_call(
        _node_dense_kernel, grid=grid, in_specs=in_specs, out_specs=out_specs,
        out_shape=out_shape,
    )(x, global_state, origin_x, *weights)


def _edge_dense(edge_attr, edge_out, Wneg, bneg, Wedge, We, be):
    grid = (pl.cdiv(E, _EDGE_BLK),)
    out_shape = (
        jax.ShapeDtypeStruct((E, OUT_CH), jnp.float32),
        jax.ShapeDtypeStruct((E, EDGE_DIM), jnp.float32),
        jax.ShapeDtypeStruct((E, EDGE_DIM), jnp.float32),
    )
    weights = (Wneg[IN_CH + STATE:], bneg.reshape(1, -1), Wedge[:EDGE_DIM], We,
               be.reshape(1, -1))
    in_specs = [
        _row_spec(_EDGE_BLK, EDGE_DIM),
        _row_spec(_EDGE_BLK, EDGE_DIM),
    ] + [_full_spec(w) for w in weights]
    out_specs = (
        _row_spec(_EDGE_BLK, OUT_CH),
        _row_spec(_EDGE_BLK, EDGE_DIM),
        _row_spec(_EDGE_BLK, EDGE_DIM),
    )
    return pl.pallas_call(
        _edge_dense_kernel, grid=grid, in_specs=in_specs, out_specs=out_specs,
        out_shape=out_shape,
    )(edge_attr, edge_out, *weights)


def kernel(x, edge_index, edge_attr, global_state, edge_out, origin_x,
           atom_index, e_idx,
           Wg1, bg1, Wg2, bg2, Wg3, bg3, Wneg, bneg, Wroot, broot,
           Wedge, bedge, We, be):
    gs, node_proj, root, oc_proj = _node_dense(
        x, global_state, origin_x, Wg1, bg1, Wg2, bg2, Wg3, bg3,
        Wneg, bneg, Wroot, broot, Wedge, bedge)
    ea_proj, eo_proj, ee = _edge_dense(edge_attr, edge_out, Wneg, bneg, Wedge,
                                       We, be)

    msg = _elu(node_proj[edge_index[1]] + ea_proj)
    x_out = root + jax.ops.segment_sum(msg, edge_index[0], num_segments=N)

    emsg = _elu(eo_proj[e_idx[1]] + oc_proj[atom_index[0]])
    edge_out2 = ee + jax.ops.segment_sum(emsg, e_idx[0], num_segments=E)
    return (x_out, gs, edge_out2)


# speedup vs baseline: 2.3854x; 1.7421x over previous
import functools

import jax
import jax.numpy as jnp
from jax import lax
from jax.experimental import pallas as pl
from jax.experimental.pallas import tpu as pltpu
from jax.experimental.pallas import tpu_sc as plsc

N = 10000
E = 320000
IN_CH = 128
OUT_CH = 128
EDGE_DIM = 48
STATE = 5

_NODE_BLK = 1024
_EDGE_BLK = 4096


def _elu(v):
    return jnp.where(v > 0, v, jnp.exp(jnp.minimum(v, 0.0)) - 1.0)


def _node_dense_kernel(x_ref, gstate_ref, ox_ref,
                       Wg1_ref, bg1_ref, Wg2_ref, bg2_ref, Wg3_ref, bg3_ref,
                       Wneg_x_ref, Wneg_g_ref, Wroot_x_ref, Wroot_g_ref, broot_ref,
                       Wedge_ox_ref, Wedge_g_ref, bedge_ref,
                       gs_out_ref, node_proj_ref, root_ref, oc_proj_ref):
    gstate = gstate_ref[...]
    h = jnp.maximum(gstate @ Wg1_ref[...] + bg1_ref[...], 0.0)
    h = jnp.maximum(h @ Wg2_ref[...] + bg2_ref[...], 0.0)
    gs = h @ Wg3_ref[...] + bg3_ref[...]
    gs_out_ref[...] = gs
    x = x_ref[...]
    node_proj_ref[...] = x @ Wneg_x_ref[...] + gs @ Wneg_g_ref[...]
    root_ref[...] = _elu(x @ Wroot_x_ref[...] + gs @ Wroot_g_ref[...] + broot_ref[...])
    oc_proj_ref[...] = (ox_ref[...] @ Wedge_ox_ref[...] + gs @ Wedge_g_ref[...]
                        + bedge_ref[...])


def _edge_dense_kernel(eo_ref, Wedge_eo_ref, We_ref, be_ref,
                       eo_proj_ref, ee_ref):
    eo = eo_ref[...]
    eo_proj_ref[...] = eo @ Wedge_eo_ref[...]
    ee_ref[...] = _elu(eo @ We_ref[...] + be_ref[...])


def _row_spec(blk, ncols):
    return pl.BlockSpec((blk, ncols), lambda i: (i, 0))


def _full_spec(arr):
    return pl.BlockSpec(arr.shape, lambda i: (0,) * arr.ndim)


def _node_dense(x, global_state, origin_x, Wg1, bg1, Wg2, bg2, Wg3, bg3,
                Wneg, bneg, Wroot, broot, Wedge, bedge):
    grid = (pl.cdiv(N, _NODE_BLK),)
    out_shape = (
        jax.ShapeDtypeStruct((_N_PAD, STATE), jnp.float32),
        jax.ShapeDtypeStruct((_N_PAD, IN_CH), jnp.float32),
        jax.ShapeDtypeStruct((_N_PAD, OUT_CH), jnp.float32),
        jax.ShapeDtypeStruct((_N_PAD, EDGE_DIM), jnp.float32),
    )
    weights = (
        Wg1, bg1.reshape(1, -1), Wg2, bg2.reshape(1, -1), Wg3, bg3.reshape(1, -1),
        Wneg[:IN_CH], Wneg[IN_CH:IN_CH + STATE],
        Wroot[:IN_CH], Wroot[IN_CH:], broot.reshape(1, -1),
        Wedge[EDGE_DIM:EDGE_DIM + IN_CH], Wedge[EDGE_DIM + IN_CH:],
        bedge.reshape(1, -1))
    in_specs = [
        _row_spec(_NODE_BLK, IN_CH),
        _row_spec(_NODE_BLK, STATE),
        _row_spec(_NODE_BLK, IN_CH),
    ] + [_full_spec(w) for w in weights]
    out_specs = (
        _row_spec(_NODE_BLK, STATE),
        _row_spec(_NODE_BLK, IN_CH),
        _row_spec(_NODE_BLK, OUT_CH),
        _row_spec(_NODE_BLK, EDGE_DIM),
    )
    return pl.pallas_call(
        _node_dense_kernel, grid=grid, in_specs=in_specs, out_specs=out_specs,
        out_shape=out_shape,
    )(x, global_state, origin_x, *weights)


def _edge_dense(edge_out, Wedge, We, be):
    grid = (pl.cdiv(E, _EDGE_BLK),)
    out_shape = (
        jax.ShapeDtypeStruct((E, EDGE_DIM), jnp.float32),
        jax.ShapeDtypeStruct((E, EDGE_DIM), jnp.float32),
    )
    weights = (Wedge[:EDGE_DIM], We, be.reshape(1, -1))
    in_specs = [_row_spec(_EDGE_BLK, EDGE_DIM),
                ] + [_full_spec(w) for w in weights]
    out_specs = (
        _row_spec(_EDGE_BLK, EDGE_DIM),
        _row_spec(_EDGE_BLK, EDGE_DIM),
    )
    return pl.pallas_call(
        _edge_dense_kernel, grid=grid, in_specs=in_specs, out_specs=out_specs,
        out_shape=out_shape,
    )(edge_out, *weights)


_SC_NC, _SC_NS = 2, 16
_EPT = E // _SC_NS
_N_PAD = 10240

_sc_mesh = plsc.VectorSubcoreMesh(core_axis_name="c", subcore_axis_name="s")


def _vperm(v, perm):
    return lax.gather(
        v, perm[:, None],
        lax.GatherDimensionNumbers(offset_dims=(), collapsed_slice_dims=(0,),
                                   start_index_map=(0,)),
        slice_sizes=(1,), mode=lax.GatherScatterMode.PROMISE_IN_BOUNDS)


def _make_node_gather(width, gb):
    scratch = [pltpu.VMEM((gb,), jnp.int32),
               pltpu.VMEM((gb, width), jnp.float32),
               pltpu.VMEM((gb,), jnp.int32),
               pltpu.VMEM((gb, width), jnp.float32),
               pltpu.SemaphoreType.DMA, pltpu.SemaphoreType.DMA]

    def body(tab_hbm, src_hbm, out_hbm, src_t0, rows_0, src_t1, rows_1,
             sem0, sem1):
        c = lax.axis_index("c")
        s = lax.axis_index("s")
        wid = s * _SC_NC + c
        ept2 = _EPT // _SC_NC
        base = wid * ept2
        nb2 = ept2 // gb
        slot0 = (src_t0, rows_0, sem0)
        slot1 = (src_t1, rows_1, sem1)

        def issue(b, slot):
            src_t, rows, sem = slot
            soff = base + b * gb
            pltpu.sync_copy(src_hbm.at[pl.ds(soff, gb)], src_t)
            pltpu.async_copy(tab_hbm.at[src_t], rows, sem)

        def flush(b, slot):
            src_t, rows, sem = slot
            pltpu.make_async_copy(tab_hbm.at[pl.ds(0, gb)], rows, sem).wait()
            pltpu.sync_copy(rows, out_hbm.at[pl.ds(base + b * gb, gb)])

        issue(0, slot0)

        def pair_body(i, carry):
            issue(2 * i + 1, slot1)
            flush(2 * i, slot0)
            issue(jnp.minimum(2 * i + 2, nb2 - 1), slot0)
            flush(2 * i + 1, slot1)
            return carry

        lax.fori_loop(0, nb2 // 2, pair_body, 0)
        pltpu.make_async_copy(tab_hbm.at[pl.ds(0, gb)], rows_0, sem0).wait()

    return functools.partial(
        pl.kernel,
        out_type=jax.ShapeDtypeStruct((E, width), jnp.float32),
        mesh=_sc_mesh,
        scratch_types=scratch,
    )(body)


def _node_msg_kernel(g_ref, ea_ref, W_ref, b_ref, msg_ref):
    msg_ref[...] = _elu(g_ref[...] + ea_ref[...] @ W_ref[...] + b_ref[...])


def _node_msg(gathered, edge_attr, Wneg_e, bneg):
    grid = (pl.cdiv(E, _EDGE_BLK),)
    weights = (Wneg_e, bneg.reshape(1, -1))
    in_specs = [_row_spec(_EDGE_BLK, IN_CH), _row_spec(_EDGE_BLK, EDGE_DIM),
                ] + [_full_spec(w) for w in weights]
    return pl.pallas_call(
        _node_msg_kernel, grid=grid, in_specs=in_specs,
        out_specs=_row_spec(_EDGE_BLK, IN_CH),
        out_shape=jax.ShapeDtypeStruct((E, IN_CH), jnp.float32),
    )(gathered, edge_attr, *weights)


def _make_node_scatter(width, n_chunks, chunk, gb):
    cps = n_chunks // _SC_NC
    stripe = chunk // _SC_NS
    nb = _EPT // gb
    assert nb % 2 == 0

    idx = lambda: pltpu.VMEM((gb,), jnp.int32)
    rows = lambda: pltpu.VMEM((gb, width), jnp.float32)
    scratch = [idx(), idx(), rows(), idx(), idx(), rows(),
               pltpu.VMEM_SHARED((chunk + 16, width), jnp.float32),
               pltpu.SemaphoreType.DMA, pltpu.SemaphoreType.DMA]

    def body(msg_hbm, dst_hbm, init_hbm, out_hbm,
             dst_t0, dl_b0, rows_0, dst_t1, dl_b1, rows_1, acc, sem0, sem1):
        c = lax.axis_index("c")
        s = lax.axis_index("s")
        iota16 = lax.iota(jnp.int32, 16)
        trash16 = chunk + iota16
        slot0 = (dst_t0, dl_b0, rows_0, sem0)
        slot1 = (dst_t1, dl_b1, rows_1, sem1)

        for p in range(cps):
            ck = c * cps + p
            lo = pl.multiple_of(ck * chunk, chunk)
            pltpu.sync_copy(
                init_hbm.at[pl.ds(lo + s * stripe, stripe)],
                acc.at[pl.ds(s * stripe, stripe)])
            plsc.subcore_barrier()

            def issue(b, slot):
                dst_t, dl_b, rows, sem = slot
                soff = s * _EPT + b * gb
                pltpu.sync_copy(dst_hbm.at[pl.ds(soff, gb)], dst_t)
                for q in range(gb // 16):
                    sl = pl.ds(q * 16, 16)
                    d = dst_t[sl]
                    m = (d >= lo) & (d < lo + chunk)
                    dl_b[sl] = jnp.where(m, d - lo, trash16)
                pltpu.async_copy(msg_hbm.at[pl.ds(soff, gb)], rows, sem)

            def scat(slot):
                dst_t, dl_b, rows, sem = slot
                pltpu.make_async_copy(msg_hbm.at[pl.ds(0, gb)], rows,
                                      sem).wait()
                pltpu.sync_copy(rows, acc.at[dl_b], add=True)

            issue(0, slot0)

            def pair_body(i, carry):
                issue(2 * i + 1, slot1)
                scat(slot0)
                issue(jnp.minimum(2 * i + 2, nb - 1), slot0)
                scat(slot1)
                return carry

            lax.fori_loop(0, nb // 2, pair_body, 0)
            pltpu.make_async_copy(msg_hbm.at[pl.ds(0, gb)], rows_0,
                                  sem0).wait()
            plsc.subcore_barrier()
            pltpu.sync_copy(
                acc.at[pl.ds(s * stripe, stripe)],
                out_hbm.at[pl.ds(lo + s * stripe, stripe)])
            plsc.subcore_barrier()

    return functools.partial(
        pl.kernel,
        out_type=jax.ShapeDtypeStruct((n_chunks * chunk, width), jnp.float32),
        mesh=_sc_mesh,
        scratch_types=scratch,
    )(body)


_node_gather = _make_node_gather(width=IN_CH, gb=200)


def _edge_msg_kernel(a_ref, b_ref, out_ref):
    out_ref[...] = _elu(a_ref[...] + b_ref[...])


def _edge_msg(a, b):
    grid = (pl.cdiv(E, _EDGE_BLK),)
    return pl.pallas_call(
        _edge_msg_kernel, grid=grid,
        in_specs=[_row_spec(_EDGE_BLK, EDGE_DIM), _row_spec(_EDGE_BLK, EDGE_DIM)],
        out_specs=_row_spec(_EDGE_BLK, EDGE_DIM),
        out_shape=jax.ShapeDtypeStruct((E, EDGE_DIM), jnp.float32),
    )(a, b)
_node_scatter = _make_node_scatter(width=IN_CH, n_chunks=2,
                                   chunk=_N_PAD // 2, gb=80)


def kernel(x, edge_index, edge_attr, global_state, edge_out, origin_x,
           atom_index, e_idx,
           Wg1, bg1, Wg2, bg2, Wg3, bg3, Wneg, bneg, Wroot, broot,
           Wedge, bedge, We, be):
    gs, node_proj, root, oc_proj = _node_dense(
        x, global_state, origin_x, Wg1, bg1, Wg2, bg2, Wg3, bg3,
        Wneg, bneg, Wroot, broot, Wedge, bedge)
    eo_proj, ee = _edge_dense(edge_out, Wedge, We, be)

    gathered = _node_gather(node_proj, edge_index[1])
    msg = _node_msg(gathered, edge_attr, Wneg[IN_CH + STATE:], bneg)
    x_out = _node_scatter(msg, edge_index[0], root)[:N]

    eo_pad = jnp.pad(eo_proj, ((0, 0), (0, IN_CH - EDGE_DIM)))
    oc_pad = jnp.pad(oc_proj, ((0, 0), (0, IN_CH - EDGE_DIM)))
    g1 = _node_gather(eo_pad, e_idx[1])[:, :EDGE_DIM]
    g2 = _node_gather(oc_pad, atom_index[0])[:, :EDGE_DIM]
    emsg = _edge_msg(g1, g2)
    edge_out2 = ee + jax.ops.segment_sum(emsg, e_idx[0], num_segments=E)
    return (x_out, gs[:N], edge_out2)
